# baseline (speedup 1.0000x reference)
import functools

import jax
import jax.numpy as jnp
from jax import lax
from jax.experimental import pallas as pl
from jax.experimental.pallas import tpu as pltpu
from jax.experimental.pallas import tpu_sc as plsc

VOCAB = 100000
EMB = 64
HID = 128
B = 1024
CTX = 20

NUM_CORES = 2
NUM_SUBCORES = 16
NW = NUM_CORES * NUM_SUBCORES
B_PER_W = B // NW
IDX_PER_W = B_PER_W * CTX
LANES = 16
EMB_CHUNKS = EMB // LANES
EMB_PAD = 128

V_TILE = 5120
N_TILES = (VOCAB + V_TILE - 1) // V_TILE


def _gather_sum_body(idx_hbm, table_hbm, out_hbm, idx_v, rows_v, acc_v, sem):
    wid = lax.axis_index("s") * NUM_CORES + lax.axis_index("c")
    base = wid * IDX_PER_W
    pltpu.sync_copy(idx_hbm.at[pl.ds(base, IDX_PER_W)], idx_v)
    pltpu.async_copy(table_hbm.at[idx_v], rows_v, sem).wait()

    def row_body(r, _):
        rb = r * CTX
        for c in range(EMB_CHUNKS):
            s = rows_v[rb, pl.ds(c * LANES, LANES)]
            for t in range(1, CTX):
                s = s + rows_v[rb + t, pl.ds(c * LANES, LANES)]
            acc_v[r, pl.ds(c * LANES, LANES)] = s
        return 0

    lax.fori_loop(0, B_PER_W, row_body, 0)
    pltpu.sync_copy(acc_v, out_hbm.at[pl.ds(wid * B_PER_W, B_PER_W)])


@functools.cache
def _gather_sum_kernel():
    return pl.kernel(
        _gather_sum_body,
        mesh=plsc.VectorSubcoreMesh(core_axis_name="c", subcore_axis_name="s"),
        out_type=jax.ShapeDtypeStruct((B, EMB_PAD), jnp.float32),
        scratch_types=[
            pltpu.VMEM((IDX_PER_W,), jnp.int32),
            pltpu.VMEM((IDX_PER_W, EMB_PAD), jnp.float32),
            pltpu.VMEM((B_PER_W, EMB_PAD), jnp.float32),
            pltpu.SemaphoreType.DMA,
        ],
    )


def _mlp_body(sum_ref, wh_ref, bh_ref, wo_ref, bo_ref, out_ref, ht_ref):
    @pl.when(pl.program_id(0) == 0)
    def _():
        avg = sum_ref[...][:, :EMB] * (1.0 / CTX)
        ht = lax.dot_general(wh_ref[...], avg, (((1,), (1,)), ((), ())),
                             preferred_element_type=jnp.float32)
        ht_ref[...] = jnp.maximum(ht + bh_ref[...], 0.0)

    logits_t = lax.dot_general(wo_ref[...], ht_ref[...],
                               (((1,), (0,)), ((), ())),
                               preferred_element_type=jnp.float32)
    bias = lax.dot_general(bo_ref[...], jnp.ones((1, B), jnp.float32),
                           (((0,), (0,)), ((), ())),
                           preferred_element_type=jnp.float32)
    out_ref[...] = logits_t + bias


def _mlp(emb_sum, W_hidden, b_hidden, W_out, b_out):
    out_t = pl.pallas_call(
        _mlp_body,
        grid=(N_TILES,),
        in_specs=[
            pl.BlockSpec((B, EMB_PAD), lambda i: (0, 0)),
            pl.BlockSpec((HID, EMB), lambda i: (0, 0)),
            pl.BlockSpec((HID, 1), lambda i: (0, 0)),
            pl.BlockSpec((V_TILE, HID), lambda i: (i, 0)),
            pl.BlockSpec((1, V_TILE), lambda i: (0, i)),
        ],
        out_specs=pl.BlockSpec((V_TILE, B), lambda i: (i, 0)),
        out_shape=jax.ShapeDtypeStruct((VOCAB, B), jnp.float32),
        scratch_shapes=[pltpu.VMEM((HID, B), jnp.float32)],
    )(emb_sum, W_hidden, b_hidden.reshape(HID, 1), W_out,
      b_out.reshape(1, VOCAB))
    return out_t.T


def kernel(context, emb_table, W_hidden, b_hidden, W_out, b_out):
    idx = context.reshape(-1).astype(jnp.int32)
    table_pad = jnp.concatenate(
        [emb_table, jnp.zeros((VOCAB, EMB_PAD - EMB), jnp.float32)], axis=1)
    emb_sum = _gather_sum_kernel()(idx, table_pad)
    return _mlp(emb_sum, W_hidden, b_hidden, W_out, b_out)


# speedup vs baseline: 1.0003x; 1.0003x over previous
import functools

import jax
import jax.numpy as jnp
from jax import lax
from jax.experimental import pallas as pl
from jax.experimental.pallas import tpu as pltpu
from jax.experimental.pallas import tpu_sc as plsc

VOCAB = 100000
EMB = 64
HID = 128
B = 1024
CTX = 20

NUM_CORES = 2
NUM_SUBCORES = 16
NW = NUM_CORES * NUM_SUBCORES
B_PER_W = B // NW
IDX_PER_W = B_PER_W * CTX
LANES = 16
EMB_CHUNKS = EMB // LANES
EMB_PAD = 128

V_TILE = 4096
N_TILES = (VOCAB + V_TILE - 1) // V_TILE


def _gather_sum_body(idx_hbm, table_hbm, out_hbm, idx_v, rows_v, acc_v, sem):
    wid = lax.axis_index("s") * NUM_CORES + lax.axis_index("c")
    base = wid * IDX_PER_W
    pltpu.sync_copy(idx_hbm.at[pl.ds(base, IDX_PER_W)], idx_v)
    pltpu.async_copy(table_hbm.at[idx_v], rows_v, sem).wait()

    def row_body(r, _):
        rb = r * CTX
        for c in range(EMB_CHUNKS):
            s = rows_v[rb, pl.ds(c * LANES, LANES)]
            for t in range(1, CTX):
                s = s + rows_v[rb + t, pl.ds(c * LANES, LANES)]
            acc_v[r, pl.ds(c * LANES, LANES)] = s
        return 0

    lax.fori_loop(0, B_PER_W, row_body, 0)
    pltpu.sync_copy(acc_v, out_hbm.at[pl.ds(wid * B_PER_W, B_PER_W)])


@functools.cache
def _gather_sum_kernel():
    return pl.kernel(
        _gather_sum_body,
        mesh=plsc.VectorSubcoreMesh(core_axis_name="c", subcore_axis_name="s"),
        out_type=jax.ShapeDtypeStruct((B, EMB_PAD), jnp.float32),
        scratch_types=[
            pltpu.VMEM((IDX_PER_W,), jnp.int32),
            pltpu.VMEM((IDX_PER_W, EMB_PAD), jnp.float32),
            pltpu.VMEM((B_PER_W, EMB_PAD), jnp.float32),
            pltpu.SemaphoreType.DMA,
        ],
    )


def _mlp_body(sum_ref, wh_ref, bh_ref, wo_ref, bo_ref, out_ref, ht_ref):
    @pl.when(pl.program_id(0) == 0)
    def _():
        avg = sum_ref[...][:, :EMB] * (1.0 / CTX)
        ht = lax.dot_general(wh_ref[...], avg, (((1,), (1,)), ((), ())),
                             preferred_element_type=jnp.float32)
        ht_ref[...] = jnp.maximum(ht + bh_ref[...], 0.0)

    logits_t = lax.dot_general(wo_ref[...], ht_ref[...],
                               (((1,), (0,)), ((), ())),
                               preferred_element_type=jnp.float32)
    bias = lax.dot_general(bo_ref[...], jnp.ones((1, B), jnp.float32),
                           (((0,), (0,)), ((), ())),
                           preferred_element_type=jnp.float32)
    out_ref[...] = logits_t + bias


def _mlp(emb_sum, W_hidden, b_hidden, W_out, b_out):
    out_t = pl.pallas_call(
        _mlp_body,
        grid=(N_TILES,),
        in_specs=[
            pl.BlockSpec((B, EMB_PAD), lambda i: (0, 0)),
            pl.BlockSpec((HID, EMB), lambda i: (0, 0)),
            pl.BlockSpec((HID, 1), lambda i: (0, 0)),
            pl.BlockSpec((V_TILE, HID), lambda i: (i, 0)),
            pl.BlockSpec((1, V_TILE), lambda i: (0, i)),
        ],
        out_specs=pl.BlockSpec((V_TILE, B), lambda i: (i, 0)),
        out_shape=jax.ShapeDtypeStruct((VOCAB, B), jnp.float32),
        scratch_shapes=[pltpu.VMEM((HID, B), jnp.float32)],
    )(emb_sum, W_hidden, b_hidden.reshape(HID, 1), W_out,
      b_out.reshape(1, VOCAB))
    return out_t.T


def kernel(context, emb_table, W_hidden, b_hidden, W_out, b_out):
    idx = context.reshape(-1).astype(jnp.int32)
    table_pad = jnp.pad(emb_table, ((0, 0), (0, EMB_PAD - EMB)))
    emb_sum = _gather_sum_kernel()(idx, table_pad)
    return _mlp(emb_sum, W_hidden, b_hidden, W_out, b_out)
